# baseline (speedup 1.0000x reference)
import functools

import jax
import jax.numpy as jnp
from jax import lax
from jax.experimental import pallas as pl
from jax.experimental.pallas import tpu as pltpu
from jax.experimental.pallas import tpu_sc as plsc

VOCAB = 100000
EMBED = 64
CTX = 10
HIDDEN = 128
NTOK = 2 * CTX
FEAT = NTOK * EMBED

NTILES = 32
VSC = 40960
RPT = VSC // NTILES
CH = 128
NCH = RPT // CH

VTC = VOCAB - VSC
BV = 8192
OFF = VSC // BV
NG = (VTC + BV - 1) // BV


def _sc_gather(emb, idx):
    mesh = plsc.VectorSubcoreMesh(core_axis_name="c", subcore_axis_name="s")

    @functools.partial(
        pl.kernel,
        mesh=mesh,
        out_type=jax.ShapeDtypeStruct((FEAT,), jnp.float32),
        scratch_types=[
            pltpu.VMEM((32,), jnp.int32),
            pltpu.VMEM((NTOK, EMBED), jnp.float32),
            pltpu.SemaphoreType.DMA,
        ],
    )
    def gather_kernel(emb_hbm, idx_hbm, out_hbm, idx_v, rows_v, sem):
        wid = lax.axis_index("s") * 2 + lax.axis_index("c")

        @pl.when(wid == 0)
        def _():
            pltpu.sync_copy(idx_hbm, idx_v.at[pl.ds(0, NTOK)])
            lo = idx_v[pl.ds(0, 16)]
            hi = idx_v[pl.ds(16, 16)]
            rows = [lo[i] for i in range(16)] + [hi[i] for i in range(NTOK - 16)]
            copies = [
                pltpu.make_async_copy(emb_hbm.at[rows[i]], rows_v.at[i], sem)
                for i in range(NTOK)
            ]
            for c in copies:
                c.start()
            for c in copies:
                c.wait()
            outs = [
                pltpu.make_async_copy(
                    rows_v.at[i], out_hbm.at[pl.ds(i * EMBED, EMBED)], sem)
                for i in range(NTOK)
            ]
            for c in outs:
                c.start()
            for c in outs:
                c.wait()

    return gather_kernel(emb, idx)


def _h_body(e_ref, w1_ref, b1_ref, h_ref):
    h2d = lax.dot_general(
        e_ref[...][None, :], w1_ref[...], (((1,), (1,)), ((), ())),
        preferred_element_type=jnp.float32)
    h_ref[...] = jnp.maximum(h2d[0] + b1_ref[...], 0.0)


def _h_tc(e, W1, b1):
    return pl.pallas_call(
        _h_body,
        out_shape=jax.ShapeDtypeStruct((HIDDEN,), jnp.float32),
    )(e, W1, b1)


def _sc_matvec(W2, h, b2):
    mesh = plsc.VectorSubcoreMesh(core_axis_name="c", subcore_axis_name="s")

    @functools.partial(
        pl.kernel,
        mesh=mesh,
        compiler_params=pltpu.CompilerParams(needs_layout_passes=False),
        out_type=jax.ShapeDtypeStruct((VSC,), jnp.float32),
        scratch_types=[
            pltpu.VMEM((CH, HIDDEN), jnp.float32),
            pltpu.VMEM((CH, HIDDEN), jnp.float32),
            pltpu.VMEM((HIDDEN,), jnp.float32),
            pltpu.VMEM((RPT,), jnp.float32),
            pltpu.SemaphoreType.DMA,
            pltpu.SemaphoreType.DMA,
        ],
    )
    def mv(w2_hbm, h_hbm, b2_hbm, out_hbm, buf_a, buf_b, h_v, o_v, sem_a, sem_b):
        wid = lax.axis_index("s") * 2 + lax.axis_index("c")
        base = wid * RPT
        pltpu.sync_copy(h_hbm, h_v)
        pltpu.sync_copy(b2_hbm.at[pl.ds(base, RPT)], o_v)
        hv = [h_v[pl.ds(16 * j, 16)] for j in range(HIDDEN // 16)]
        row_iota = lax.iota(jnp.int32, 16)
        onehot = [
            (row_iota == l).astype(jnp.float32) for l in range(16)
        ]
        perm = {
            s: jnp.bitwise_xor(row_iota, s) for s in (1, 2, 4, 8)
        }

        def start(g, buf, sem):
            pltpu.make_async_copy(
                w2_hbm.at[pl.ds(base + g * CH, CH)], buf, sem).start()

        def process(g, buf, sem):
            pltpu.make_async_copy(
                w2_hbm.at[pl.ds(base + g * CH, CH)], buf, sem).wait()
            for grp in range(CH // 16):
                terms = [o_v[pl.ds(g * CH + grp * 16, 16)]]
                for l in range(16):
                    r = grp * 16 + l
                    acc = buf[r, pl.ds(0, 16)] * hv[0]
                    for j in range(1, HIDDEN // 16):
                        acc = acc + buf[r, pl.ds(16 * j, 16)] * hv[j]
                    cs = plsc.cumsum(acc)
                    terms.append(cs[15] * onehot[l])
                while len(terms) > 1:
                    terms = [terms[i] + terms[i + 1]
                             for i in range(0, len(terms) - 1, 2)] + (
                                 [terms[-1]] if len(terms) % 2 else [])
                o_v[pl.ds(g * CH + grp * 16, 16)] = terms[0]

        start(0, buf_a, sem_a)

        def chunk(g, _):
            even = (g % 2) == 0

            @pl.when(even)
            def _():
                @pl.when(g + 1 < NCH)
                def _():
                    start(g + 1, buf_b, sem_b)
                process(g, buf_a, sem_a)

            @pl.when(jnp.logical_not(even))
            def _():
                @pl.when(g + 1 < NCH)
                def _():
                    start(g + 1, buf_a, sem_a)
                process(g, buf_b, sem_b)

            return ()

        lax.fori_loop(0, NCH, chunk, (), unroll=False)
        pltpu.sync_copy(o_v, out_hbm.at[pl.ds(base, RPT)])

    return mv(W2, h, b2)


def _tail_body(e_ref, w1_ref, b1_ref, w2_ref, b2_ref, out_ref, h_ref):
    @pl.when(pl.program_id(0) == 0)
    def _():
        h2 = lax.dot_general(
            e_ref[...][None, :], w1_ref[...], (((1,), (1,)), ((), ())),
            preferred_element_type=jnp.float32)
        h_ref[...] = jnp.maximum(h2 + b1_ref[...][None, :], 0.0)

    out2d = lax.dot_general(
        h_ref[...], w2_ref[...], (((1,), (1,)), ((), ())),
        preferred_element_type=jnp.float32)
    out_ref[...] = out2d[0] + b2_ref[...]


def _tc_matvec(e, W1, b1, W2, b2):
    return pl.pallas_call(
        _tail_body,
        grid=(NG,),
        in_specs=[
            pl.BlockSpec((FEAT,), lambda i: (0,)),
            pl.BlockSpec((HIDDEN, FEAT), lambda i: (0, 0)),
            pl.BlockSpec((HIDDEN,), lambda i: (0,)),
            pl.BlockSpec((BV, HIDDEN), lambda i: (OFF + i, 0)),
            pl.BlockSpec((BV,), lambda i: (OFF + i,)),
        ],
        out_specs=pl.BlockSpec((BV,), lambda i: (i,)),
        out_shape=jax.ShapeDtypeStruct((VTC,), jnp.float32),
        scratch_shapes=[pltpu.VMEM((1, HIDDEN), jnp.float32)],
    )(e, W1, b1, W2, b2)


def kernel(x, emb, W1, b1, W2, b2):
    e = _sc_gather(emb, x.astype(jnp.int32))
    out_tc = _tc_matvec(e, W1, b1, W2, b2)
    h = _h_tc(e, W1, b1)
    out_sc = _sc_matvec(W2, h, b2)
    out_sc, out_tc = lax.optimization_barrier((out_sc, out_tc))
    return jnp.concatenate([out_sc, out_tc])[None, :]


# speedup vs baseline: 1.2199x; 1.2199x over previous
import functools

import jax
import jax.numpy as jnp
from jax import lax
from jax.experimental import pallas as pl
from jax.experimental.pallas import tpu as pltpu
from jax.experimental.pallas import tpu_sc as plsc

VOCAB = 100000
EMBED = 64
CTX = 10
HIDDEN = 128
NTOK = 2 * CTX
FEAT = NTOK * EMBED

BV = 16384
NG = (VOCAB + BV - 1) // BV


def _sc_gather(emb, idx):
    mesh = plsc.VectorSubcoreMesh(core_axis_name="c", subcore_axis_name="s")

    @functools.partial(
        pl.kernel,
        mesh=mesh,
        out_type=jax.ShapeDtypeStruct((FEAT,), jnp.float32),
        scratch_types=[
            pltpu.VMEM((32,), jnp.int32),
            pltpu.VMEM((NTOK, EMBED), jnp.float32),
            pltpu.SemaphoreType.DMA,
        ],
    )
    def gather_kernel(emb_hbm, idx_hbm, out_hbm, idx_v, rows_v, sem):
        wid = lax.axis_index("s") * 2 + lax.axis_index("c")

        @pl.when(wid == 0)
        def _():
            pltpu.sync_copy(idx_hbm, idx_v.at[pl.ds(0, NTOK)])
            lo = idx_v[pl.ds(0, 16)]
            hi = idx_v[pl.ds(16, 16)]
            rows = [lo[i] for i in range(16)] + [hi[i] for i in range(NTOK - 16)]
            copies = [
                pltpu.make_async_copy(emb_hbm.at[rows[i]], rows_v.at[i], sem)
                for i in range(NTOK)
            ]
            for c in copies:
                c.start()
            for c in copies:
                c.wait()
            outs = [
                pltpu.make_async_copy(
                    rows_v.at[i], out_hbm.at[pl.ds(i * EMBED, EMBED)], sem)
                for i in range(NTOK)
            ]
            for c in outs:
                c.start()
            for c in outs:
                c.wait()

    return gather_kernel(emb, idx)


def _mlp_body(e_ref, w1_ref, b1_ref, w2_ref, b2_ref, out_ref, h_ref):
    @pl.when(pl.program_id(0) == 0)
    def _():
        h2 = lax.dot_general(
            e_ref[...][None, :], w1_ref[...], (((1,), (1,)), ((), ())),
            preferred_element_type=jnp.float32)
        h_ref[...] = jnp.maximum(h2 + b1_ref[...][None, :], 0.0)

    out2 = lax.dot_general(
        h_ref[...], w2_ref[...], (((1,), (1,)), ((), ())),
        preferred_element_type=jnp.float32)
    out_ref[...] = out2[0] + b2_ref[...]


def _tc_mlp(e, W1, b1, W2, b2):
    return pl.pallas_call(
        _mlp_body,
        grid=(NG,),
        in_specs=[
            pl.BlockSpec((FEAT,), lambda i: (0,)),
            pl.BlockSpec((HIDDEN, FEAT), lambda i: (0, 0)),
            pl.BlockSpec((HIDDEN,), lambda i: (0,)),
            pl.BlockSpec((BV, HIDDEN), lambda i: (i, 0)),
            pl.BlockSpec((BV,), lambda i: (i,)),
        ],
        out_specs=pl.BlockSpec((BV,), lambda i: (i,)),
        out_shape=jax.ShapeDtypeStruct((VOCAB,), jnp.float32),
        scratch_shapes=[pltpu.VMEM((1, HIDDEN), jnp.float32)],
    )(e, W1, b1, W2, b2)


def kernel(x, emb, W1, b1, W2, b2):
    e = _sc_gather(emb, x.astype(jnp.int32))
    return _tc_mlp(e, W1, b1, W2, b2)[None, :]


# speedup vs baseline: 1.2614x; 1.0340x over previous
import functools

import jax
import jax.numpy as jnp
from jax import lax
from jax.experimental import pallas as pl
from jax.experimental.pallas import tpu as pltpu
from jax.experimental.pallas import tpu_sc as plsc

VOCAB = 100000
EMBED = 64
CTX = 10
HIDDEN = 128
NTOK = 2 * CTX
FEAT = NTOK * EMBED

BV = 16384
NG = (VOCAB + BV - 1) // BV


def _sc_gather(emb, idx):
    mesh = plsc.VectorSubcoreMesh(core_axis_name="c", subcore_axis_name="s")

    @functools.partial(
        pl.kernel,
        mesh=mesh,
        out_type=jax.ShapeDtypeStruct((FEAT,), jnp.float32),
        scratch_types=[
            pltpu.VMEM((32,), jnp.int32),
            pltpu.VMEM((NTOK, EMBED), jnp.float32),
            pltpu.SemaphoreType.DMA,
        ],
    )
    def gather_kernel(emb_hbm, idx_hbm, out_hbm, idx_v, rows_v, sem):
        wid = lax.axis_index("s") * 2 + lax.axis_index("c")

        @pl.when(wid == 0)
        def _():
            pltpu.sync_copy(idx_hbm, idx_v.at[pl.ds(0, NTOK)])
            lo = idx_v[pl.ds(0, 16)]
            hi = idx_v[pl.ds(16, 16)]
            rows = [lo[i] for i in range(16)] + [hi[i] for i in range(NTOK - 16)]
            copies = [
                pltpu.make_async_copy(emb_hbm.at[rows[i]], rows_v.at[i], sem)
                for i in range(NTOK)
            ]
            for c in copies:
                c.start()
            for c in copies:
                c.wait()
            outs = [
                pltpu.make_async_copy(
                    rows_v.at[i], out_hbm.at[pl.ds(i * EMBED, EMBED)], sem)
                for i in range(NTOK)
            ]
            for c in outs:
                c.start()
            for c in outs:
                c.wait()

    return gather_kernel(emb, idx)


def _mlp_body(e_ref, w1_ref, b1_ref, w2_ref, b2_ref, out_ref, h_ref):
    @pl.when(pl.program_id(0) == 0)
    def _():
        h2 = lax.dot_general(
            e_ref[...][None, :], w1_ref[...], (((1,), (1,)), ((), ())),
            preferred_element_type=jnp.float32)
        h_ref[...] = jnp.maximum(h2 + b1_ref[...][None, :], 0.0)

    out2 = lax.dot_general(
        h_ref[...], w2_ref[...], (((1,), (1,)), ((), ())),
        preferred_element_type=jnp.float32)
    out_ref[...] = out2 + b2_ref[...][None, :]


def _tc_mlp(e, W1, b1, W2, b2):
    return pl.pallas_call(
        _mlp_body,
        grid=(NG,),
        in_specs=[
            pl.BlockSpec((FEAT,), lambda i: (0,)),
            pl.BlockSpec((HIDDEN, FEAT), lambda i: (0, 0)),
            pl.BlockSpec((HIDDEN,), lambda i: (0,)),
            pl.BlockSpec((BV, HIDDEN), lambda i: (i, 0)),
            pl.BlockSpec((BV,), lambda i: (i,)),
        ],
        out_specs=pl.BlockSpec((1, BV), lambda i: (0, i)),
        out_shape=jax.ShapeDtypeStruct((1, VOCAB), jnp.float32),
        scratch_shapes=[pltpu.VMEM((1, HIDDEN), jnp.float32)],
    )(e, W1, b1, W2, b2)


def kernel(x, emb, W1, b1, W2, b2):
    e = _sc_gather(emb, x.astype(jnp.int32))
    return _tc_mlp(e, W1, b1, W2, b2)
